# speedup vs baseline: 1.1045x; 1.1045x over previous
import functools

import jax
import jax.numpy as jnp
from jax import lax
from jax.experimental import pallas as pl
from jax.experimental.pallas import tpu as pltpu
from jax.experimental.pallas import tpu_sc as plsc

_NUM_CORES = 2
_NUM_SUBCORES = 16
_NW = _NUM_CORES * _NUM_SUBCORES


@functools.lru_cache(maxsize=None)
def _make_gather(table_len: int, batch: int):
    assert batch % (8 * _NW) == 0
    b_per_w = batch // _NW
    mesh = plsc.VectorSubcoreMesh(core_axis_name="c", subcore_axis_name="s")

    @functools.partial(
        pl.kernel,
        mesh=mesh,
        out_type=jax.ShapeDtypeStruct((batch,), jnp.float32),
        scratch_types=[
            pltpu.VMEM((b_per_w,), jnp.int32),
            pltpu.VMEM((b_per_w,), jnp.float32),
            pltpu.SemaphoreType.DMA,
        ],
    )
    def gather_kernel(idx_hbm, table_hbm, out_hbm, idx_v, vals_v, sem):
        wid = lax.axis_index("s") * _NUM_CORES + lax.axis_index("c")
        base = wid * b_per_w
        pltpu.sync_copy(idx_hbm.at[pl.ds(base, b_per_w)], idx_v)
        pltpu.async_copy(table_hbm.at[idx_v], vals_v, sem).wait()
        pltpu.sync_copy(vals_v, out_hbm.at[pl.ds(base, b_per_w)])

    return gather_kernel


def kernel(user_ids, item_ids, items_cnts):
    del user_ids
    return _make_gather(items_cnts.shape[0], item_ids.shape[0])(
        item_ids, items_cnts
    )


# speedup vs baseline: 1.1129x; 1.0077x over previous
import functools

import jax
import jax.numpy as jnp
from jax import lax
from jax.experimental import pallas as pl
from jax.experimental.pallas import tpu as pltpu
from jax.experimental.pallas import tpu_sc as plsc

_NUM_CORES = 2
_NUM_SUBCORES = 16
_NW = _NUM_CORES * _NUM_SUBCORES


@functools.lru_cache(maxsize=None)
def _make_gather(table_len: int, batch: int):
    assert batch % (8 * _NW) == 0
    b_per_w = batch // _NW
    mesh = plsc.VectorSubcoreMesh(core_axis_name="c", subcore_axis_name="s")

    n_chunks = 4
    chunk = b_per_w // n_chunks
    assert chunk % 8 == 0

    @functools.partial(
        pl.kernel,
        mesh=mesh,
        out_type=jax.ShapeDtypeStruct((batch,), jnp.float32),
        scratch_types=[
            pltpu.VMEM((b_per_w,), jnp.int32),
            pltpu.VMEM((b_per_w,), jnp.float32),
        ]
        + [pltpu.SemaphoreType.DMA] * n_chunks
        + [pltpu.SemaphoreType.DMA],
    )
    def gather_kernel(idx_hbm, table_hbm, out_hbm, idx_v, vals_v, *sems):
        gsems, osem = sems[:n_chunks], sems[n_chunks]
        wid = lax.axis_index("s") * _NUM_CORES + lax.axis_index("c")
        base = wid * b_per_w
        pltpu.sync_copy(idx_hbm.at[pl.ds(base, b_per_w)], idx_v)
        gathers = []
        for c in range(n_chunks):
            sl = pl.ds(c * chunk, chunk)
            gathers.append(
                pltpu.async_copy(
                    table_hbm.at[idx_v.at[sl]], vals_v.at[sl], gsems[c]
                )
            )
        stores = []
        for c in range(n_chunks):
            gathers[c].wait()
            stores.append(
                pltpu.async_copy(
                    vals_v.at[pl.ds(c * chunk, chunk)],
                    out_hbm.at[pl.ds(base + c * chunk, chunk)],
                    osem,
                )
            )
        for st in stores:
            st.wait()

    return gather_kernel


def kernel(user_ids, item_ids, items_cnts):
    del user_ids
    return _make_gather(items_cnts.shape[0], item_ids.shape[0])(
        item_ids, items_cnts
    )


# speedup vs baseline: 1.1224x; 1.0085x over previous
import functools

import jax
import jax.numpy as jnp
from jax import lax
from jax.experimental import pallas as pl
from jax.experimental.pallas import tpu as pltpu
from jax.experimental.pallas import tpu_sc as plsc

_NUM_CORES = 2
_NUM_SUBCORES = 16
_NW = _NUM_CORES * _NUM_SUBCORES


@functools.lru_cache(maxsize=None)
def _make_gather(table_len: int, batch: int):
    assert batch % (8 * _NW) == 0
    b_per_w = batch // _NW
    mesh = plsc.VectorSubcoreMesh(core_axis_name="c", subcore_axis_name="s")

    n_chunks = 4
    chunk = b_per_w // n_chunks
    assert chunk % 8 == 0

    @functools.partial(
        pl.kernel,
        mesh=mesh,
        out_type=jax.ShapeDtypeStruct((batch,), jnp.float32),
        scratch_types=[
            pltpu.VMEM((b_per_w,), jnp.int32),
            pltpu.VMEM((b_per_w,), jnp.float32),
        ]
        + [pltpu.SemaphoreType.DMA] * (2 * n_chunks)
        + [pltpu.SemaphoreType.DMA],
    )
    def gather_kernel(idx_hbm, table_hbm, out_hbm, idx_v, vals_v, *sems):
        isems = sems[:n_chunks]
        gsems = sems[n_chunks : 2 * n_chunks]
        osem = sems[2 * n_chunks]
        wid = lax.axis_index("s") * _NUM_CORES + lax.axis_index("c")
        base = wid * b_per_w
        loads = []
        for c in range(n_chunks):
            sl = pl.ds(c * chunk, chunk)
            loads.append(
                pltpu.async_copy(
                    idx_hbm.at[pl.ds(base + c * chunk, chunk)],
                    idx_v.at[sl],
                    isems[c],
                )
            )
        gathers = []
        for c in range(n_chunks):
            sl = pl.ds(c * chunk, chunk)
            loads[c].wait()
            gathers.append(
                pltpu.async_copy(
                    table_hbm.at[idx_v.at[sl]], vals_v.at[sl], gsems[c]
                )
            )
        stores = []
        for c in range(n_chunks):
            gathers[c].wait()
            stores.append(
                pltpu.async_copy(
                    vals_v.at[pl.ds(c * chunk, chunk)],
                    out_hbm.at[pl.ds(base + c * chunk, chunk)],
                    osem,
                )
            )
        for st in stores:
            st.wait()

    return gather_kernel


def kernel(user_ids, item_ids, items_cnts):
    del user_ids
    return _make_gather(items_cnts.shape[0], item_ids.shape[0])(
        item_ids, items_cnts
    )
